# baseline (speedup 1.0000x reference)
import functools

import jax
import jax.numpy as jnp
from jax import lax
from jax.experimental import pallas as pl
from jax.experimental.pallas import tpu as pltpu
from jax.experimental.pallas import tpu_sc as plsc

N = 10000
E = 160000
F = 256
HALF = 128
G = 64
C = 32

NC = 2
NS = 16
CHUNK = 128
E_TILE = E // NS
NCHUNK = 80
E_TILE_PAD = NCHUNK * CHUNK
NPHASE = 2
CPP = NCHUNK // NPHASE
N_PAD = 10240
ROWS_PER_TILE = N_PAD // NS


def _sc_body(xh, srcs, dsts, out, src_v, dst_v, buf0, buf1, acc, sem0, sem1):
    cid = lax.axis_index("c")
    sid = lax.axis_index("s")

    def _zero_row(r, carry):
        for k in range(HALF // 16):
            buf0[r, pl.ds(k * 16, 16)] = jnp.zeros((16,), jnp.float32)
        return carry

    lax.fori_loop(0, CHUNK, _zero_row, 0)
    for b in range(ROWS_PER_TILE // CHUNK):
        pltpu.sync_copy(buf0, acc.at[pl.ds(sid * ROWS_PER_TILE + b * CHUNK, CHUNK)])

    plsc.subcore_barrier()

    for ph in range(NPHASE):
        pltpu.sync_copy(srcs.at[sid, ph], src_v)
        pltpu.sync_copy(dsts.at[sid, ph], dst_v)

        def _fix_row(j, carry):
            for k in range(CHUNK // 16):
                v = src_v[j, pl.ds(k * 16, 16)]
                src_v[j, pl.ds(k * 16, 16)] = v + cid * N
            return carry

        lax.fori_loop(0, CPP, _fix_row, 0)
        pltpu.async_copy(xh.at[src_v.at[0]], buf0, sem0)

        def _step(k, carry):
            j0 = 2 * k
            pltpu.make_async_copy(xh.at[src_v.at[0]], buf0, sem0).wait()
            pltpu.async_copy(xh.at[src_v.at[j0 + 1]], buf1, sem1)
            pltpu.sync_copy(buf0, acc.at[dst_v.at[j0]], add=True)
            pltpu.make_async_copy(xh.at[src_v.at[0]], buf1, sem1).wait()
            j2 = jnp.minimum(j0 + 2, CPP - 1)
            pltpu.async_copy(xh.at[src_v.at[j2]], buf0, sem0)
            pltpu.sync_copy(buf1, acc.at[dst_v.at[j0 + 1]], add=True)
            return carry

        lax.fori_loop(0, CPP // 2, _step, 0)
        pltpu.make_async_copy(xh.at[src_v.at[0]], buf0, sem0).wait()

    plsc.subcore_barrier()

    pltpu.sync_copy(acc.at[pl.ds(sid * ROWS_PER_TILE, ROWS_PER_TILE)],
                    out.at[cid, pl.ds(sid * ROWS_PER_TILE, ROWS_PER_TILE)])


@functools.cache
def _sc_scatter():
    return pl.kernel(
        _sc_body,
        out_type=jax.ShapeDtypeStruct((NC, N_PAD, HALF), jnp.float32),
        mesh=plsc.VectorSubcoreMesh(core_axis_name="c", subcore_axis_name="s",
                                    num_cores=NC, num_subcores=NS),
        scratch_types=[
            pltpu.VMEM((CPP, CHUNK), jnp.int32),
            pltpu.VMEM((CPP, CHUNK), jnp.int32),
            pltpu.VMEM((CHUNK, HALF), jnp.float32),
            pltpu.VMEM((CHUNK, HALF), jnp.float32),
            pltpu.VMEM_SHARED((N_PAD, HALF), jnp.float32),
            pltpu.SemaphoreType.DMA,
            pltpu.SemaphoreType.DMA,
        ],
    )


NB = 10
BN = N // NB


def _tc_body(agg0, agg1, x, batch, wrt0, wrt1, wroott, brel, wfct, bfc,
             out, acc, cnt):
    i = pl.program_id(0)

    @pl.when(i == 0)
    def _():
        acc[...] = jnp.zeros_like(acc)
        cnt[...] = jnp.zeros_like(cnt)

    h = (jnp.dot(agg0[0], wrt0[...], preferred_element_type=jnp.float32)
         + jnp.dot(agg1[0], wrt1[...], preferred_element_type=jnp.float32)
         + jnp.dot(x[...], wroott[...], preferred_element_type=jnp.float32)
         + brel[...])
    h = jnp.where(h >= 0, h, 0.01 * h)

    b = batch[0, 0, :]
    gids = lax.broadcasted_iota(jnp.int32, (G, BN), 0)
    m = (b[None, :] == gids).astype(jnp.float32)
    acc[...] = acc[...] + jnp.dot(m, h, preferred_element_type=jnp.float32)
    cnt[...] = cnt[...] + jnp.sum(m, axis=1, keepdims=True)

    @pl.when(i == NB - 1)
    def _():
        cv = cnt[...]
        c2 = jnp.maximum(jnp.concatenate([cv, cv], axis=1), 1.0)
        pooled = acc[...] / c2
        out[...] = (jnp.dot(pooled, wfct[...], preferred_element_type=jnp.float32)
                    + bfc[...])


_tc_dense = pl.pallas_call(
    _tc_body,
    grid=(NB,),
    in_specs=[
        pl.BlockSpec((1, BN, HALF), lambda i: (0, i, 0)),
        pl.BlockSpec((1, BN, HALF), lambda i: (1, i, 0)),
        pl.BlockSpec((BN, F), lambda i: (i, 0)),
        pl.BlockSpec((1, 1, BN), lambda i: (i, 0, 0)),
        pl.BlockSpec((HALF, F), lambda i: (0, 0)),
        pl.BlockSpec((HALF, F), lambda i: (0, 0)),
        pl.BlockSpec((F, F), lambda i: (0, 0)),
        pl.BlockSpec((1, F), lambda i: (0, 0)),
        pl.BlockSpec((F, C), lambda i: (0, 0)),
        pl.BlockSpec((1, C), lambda i: (0, 0)),
    ],
    out_specs=pl.BlockSpec((G, C), lambda i: (0, 0)),
    out_shape=jax.ShapeDtypeStruct((G, C), jnp.float32),
    scratch_shapes=[
        pltpu.VMEM((G, F), jnp.float32),
        pltpu.VMEM((G, HALF), jnp.float32),
    ],
)


def kernel(x, edge_index, batch, W_rel, b_rel, W_root, W_fc, b_fc):
    src = edge_index[0].astype(jnp.int32)
    dst = edge_index[1].astype(jnp.int32)

    pad = E_TILE_PAD - E_TILE
    src_t = jnp.concatenate(
        [src.reshape(NS, E_TILE), jnp.zeros((NS, pad), jnp.int32)], axis=1)
    dst_t = jnp.concatenate(
        [dst.reshape(NS, E_TILE), jnp.full((NS, pad), N, jnp.int32)], axis=1)
    src3 = src_t.reshape(NS, NPHASE, CPP, CHUNK)
    dst3 = dst_t.reshape(NS, NPHASE, CPP, CHUNK)
    xh = jnp.concatenate([x[:, :HALF], x[:, HALF:]], axis=0)

    agg = _sc_scatter()(xh, src3, dst3)

    batch3 = batch.astype(jnp.int32).reshape(NB, 1, BN)
    return _tc_dense(
        agg, agg, x, batch3,
        W_rel.T[:HALF].astype(jnp.float32),
        W_rel.T[HALF:].astype(jnp.float32),
        W_root.T.astype(jnp.float32),
        b_rel.reshape(1, F),
        W_fc.T.astype(jnp.float32),
        b_fc.reshape(1, C),
    )


# speedup vs baseline: 1.3785x; 1.3785x over previous
import functools

import jax
import jax.numpy as jnp
from jax import lax
from jax.experimental import pallas as pl
from jax.experimental.pallas import tpu as pltpu
from jax.experimental.pallas import tpu_sc as plsc

N = 10000
E = 160000
F = 256
HALF = 128
G = 64
C = 32

NC = 2
NS = 16
CHUNK = 128
E_TILE = E // NS
NCHUNK = -(-E_TILE // CHUNK)
E_TILE_PAD = NCHUNK * CHUNK
N_PAD = 10240
ROWS_PER_TILE = N_PAD // NS


def _sc_body(xh, srcs, dsts, out, src_v, dst_v, buf0, acc, sem0):
    cid = lax.axis_index("c")
    sid = lax.axis_index("s")

    def _zero_row(r, carry):
        for k in range(HALF // 16):
            buf0[r, pl.ds(k * 16, 16)] = jnp.zeros((16,), jnp.float32)
        return carry

    lax.fori_loop(0, CHUNK, _zero_row, 0)
    for b in range(ROWS_PER_TILE // CHUNK):
        pltpu.sync_copy(buf0, acc.at[pl.ds(sid * ROWS_PER_TILE + b * CHUNK, CHUNK)])

    pltpu.sync_copy(srcs.at[cid, sid], src_v)
    pltpu.sync_copy(dsts.at[sid], dst_v)
    plsc.subcore_barrier()

    def _step(j, carry):
        pltpu.async_copy(xh.at[src_v.at[j]], buf0, sem0).wait()
        pltpu.sync_copy(buf0, acc.at[dst_v.at[j]], add=True)
        return carry

    lax.fori_loop(0, NCHUNK, _step, 0)
    plsc.subcore_barrier()

    pltpu.sync_copy(acc.at[pl.ds(sid * ROWS_PER_TILE, ROWS_PER_TILE)],
                    out.at[cid, pl.ds(sid * ROWS_PER_TILE, ROWS_PER_TILE)])


@functools.cache
def _sc_scatter():
    return pl.kernel(
        _sc_body,
        out_type=jax.ShapeDtypeStruct((NC, N_PAD, HALF), jnp.float32),
        mesh=plsc.VectorSubcoreMesh(core_axis_name="c", subcore_axis_name="s",
                                    num_cores=NC, num_subcores=NS),
        scratch_types=[
            pltpu.VMEM((NCHUNK, CHUNK), jnp.int32),
            pltpu.VMEM((NCHUNK, CHUNK), jnp.int32),
            pltpu.VMEM((CHUNK, HALF), jnp.float32),
            pltpu.VMEM_SHARED((N_PAD, HALF), jnp.float32),
            pltpu.SemaphoreType.DMA,
        ],
    )


NB = 10
BN = N // NB


def _tc_body(agg0, agg1, x, batch, wrt0, wrt1, wroott, brel, wfct, bfc,
             out, acc, cnt):
    i = pl.program_id(0)

    @pl.when(i == 0)
    def _():
        acc[...] = jnp.zeros_like(acc)
        cnt[...] = jnp.zeros_like(cnt)

    h = (jnp.dot(agg0[0], wrt0[...], preferred_element_type=jnp.float32)
         + jnp.dot(agg1[0], wrt1[...], preferred_element_type=jnp.float32)
         + jnp.dot(x[...], wroott[...], preferred_element_type=jnp.float32)
         + brel[...])
    h = jnp.where(h >= 0, h, 0.01 * h)

    b = batch[0, 0, :]
    gids = lax.broadcasted_iota(jnp.int32, (G, BN), 0)
    m = (b[None, :] == gids).astype(jnp.float32)
    acc[...] = acc[...] + jnp.dot(m, h, preferred_element_type=jnp.float32)
    cnt[...] = cnt[...] + jnp.sum(m, axis=1, keepdims=True)

    @pl.when(i == NB - 1)
    def _():
        cv = cnt[...]
        c2 = jnp.maximum(jnp.concatenate([cv, cv], axis=1), 1.0)
        pooled = acc[...] / c2
        out[...] = (jnp.dot(pooled, wfct[...], preferred_element_type=jnp.float32)
                    + bfc[...])


_tc_dense = pl.pallas_call(
    _tc_body,
    grid=(NB,),
    in_specs=[
        pl.BlockSpec((1, BN, HALF), lambda i: (0, i, 0)),
        pl.BlockSpec((1, BN, HALF), lambda i: (1, i, 0)),
        pl.BlockSpec((BN, F), lambda i: (i, 0)),
        pl.BlockSpec((1, 1, BN), lambda i: (i, 0, 0)),
        pl.BlockSpec((HALF, F), lambda i: (0, 0)),
        pl.BlockSpec((HALF, F), lambda i: (0, 0)),
        pl.BlockSpec((F, F), lambda i: (0, 0)),
        pl.BlockSpec((1, F), lambda i: (0, 0)),
        pl.BlockSpec((F, C), lambda i: (0, 0)),
        pl.BlockSpec((1, C), lambda i: (0, 0)),
    ],
    out_specs=pl.BlockSpec((G, C), lambda i: (0, 0)),
    out_shape=jax.ShapeDtypeStruct((G, C), jnp.float32),
    scratch_shapes=[
        pltpu.VMEM((G, F), jnp.float32),
        pltpu.VMEM((G, HALF), jnp.float32),
    ],
)


def kernel(x, edge_index, batch, W_rel, b_rel, W_root, W_fc, b_fc):
    src = edge_index[0].astype(jnp.int32)
    dst = edge_index[1].astype(jnp.int32)

    pad = E_TILE_PAD - E_TILE
    src_t = jnp.concatenate(
        [src.reshape(NS, E_TILE), jnp.zeros((NS, pad), jnp.int32)], axis=1)
    dst_t = jnp.concatenate(
        [dst.reshape(NS, E_TILE), jnp.full((NS, pad), N, jnp.int32)], axis=1)
    src3 = jnp.stack([src_t, src_t + N]).reshape(NC, NS, NCHUNK, CHUNK)
    dst3 = dst_t.reshape(NS, NCHUNK, CHUNK)
    xh = jnp.concatenate([x[:, :HALF], x[:, HALF:]], axis=0)

    agg = _sc_scatter()(xh, src3, dst3)

    batch3 = batch.astype(jnp.int32).reshape(NB, 1, BN)
    return _tc_dense(
        agg, agg, x, batch3,
        W_rel.T[:HALF].astype(jnp.float32),
        W_rel.T[HALF:].astype(jnp.float32),
        W_root.T.astype(jnp.float32),
        b_rel.reshape(1, F),
        W_fc.T.astype(jnp.float32),
        b_fc.reshape(1, C),
    )


# speedup vs baseline: 1.4421x; 1.0462x over previous
import functools

import jax
import jax.numpy as jnp
from jax import lax
from jax.experimental import pallas as pl
from jax.experimental.pallas import tpu as pltpu
from jax.experimental.pallas import tpu_sc as plsc

N = 10000
E = 160000
F = 256
HALF = 128
G = 64
C = 32

NC = 2
NS = 16
CHUNK = 128
E_TILE = E // NS
NCHUNK = -(-E_TILE // CHUNK)
E_TILE_PAD = NCHUNK * CHUNK
N_PAD = 10240
ROWS_PER_TILE = N_PAD // NS


def _sc_body(xh, srcs, dsts, out, src_v, dst_v, buf0, acc, sem0):
    cid = lax.axis_index("c")
    sid = lax.axis_index("s")

    def _zero_row(r, carry):
        for k in range(HALF // 16):
            buf0[r, pl.ds(k * 16, 16)] = jnp.zeros((16,), jnp.float32)
        return carry

    lax.fori_loop(0, CHUNK, _zero_row, 0)
    for b in range(ROWS_PER_TILE // CHUNK):
        pltpu.sync_copy(buf0, acc.at[pl.ds(sid * ROWS_PER_TILE + b * CHUNK, CHUNK)])

    pltpu.sync_copy(srcs.at[cid, sid], src_v)
    pltpu.sync_copy(dsts.at[sid], dst_v)
    plsc.subcore_barrier()

    def _step(j, carry):
        pltpu.async_copy(xh.at[src_v.at[j]], buf0, sem0).wait()
        pltpu.sync_copy(buf0, acc.at[dst_v.at[j]], add=True)
        return carry

    lax.fori_loop(0, NCHUNK, _step, 0)
    plsc.subcore_barrier()

    pltpu.sync_copy(acc.at[pl.ds(sid * ROWS_PER_TILE, ROWS_PER_TILE)],
                    out.at[cid, pl.ds(sid * ROWS_PER_TILE, ROWS_PER_TILE)])


@functools.cache
def _sc_scatter():
    return pl.kernel(
        _sc_body,
        out_type=jax.ShapeDtypeStruct((NC, N_PAD, HALF), jnp.float32),
        mesh=plsc.VectorSubcoreMesh(core_axis_name="c", subcore_axis_name="s",
                                    num_cores=NC, num_subcores=NS),
        scratch_types=[
            pltpu.VMEM((NCHUNK, CHUNK), jnp.int32),
            pltpu.VMEM((NCHUNK, CHUNK), jnp.int32),
            pltpu.VMEM((CHUNK, HALF), jnp.float32),
            pltpu.VMEM_SHARED((N_PAD, HALF), jnp.float32),
            pltpu.SemaphoreType.DMA,
        ],
    )


NB = 10
BN = N // NB


def _tc_body(agg0, agg1, x, batch, wrt0, wrt1, wroott, brel, wfct, bfc,
             out, acc, cnt):
    i = pl.program_id(0)

    @pl.when(i == 0)
    def _():
        acc[...] = jnp.zeros_like(acc)
        cnt[...] = jnp.zeros_like(cnt)

    h = (jnp.dot(agg0[0], wrt0[...], preferred_element_type=jnp.float32)
         + jnp.dot(agg1[0], wrt1[...], preferred_element_type=jnp.float32)
         + jnp.dot(x[...], wroott[...], preferred_element_type=jnp.float32)
         + brel[...])
    h = jnp.where(h >= 0, h, 0.01 * h)

    b = batch[0, 0, :]
    gids = lax.broadcasted_iota(jnp.int32, (G, BN), 0)
    m = (b[None, :] == gids).astype(jnp.float32)
    acc[...] = acc[...] + jnp.dot(m, h, preferred_element_type=jnp.float32)
    cnt[...] = cnt[...] + jnp.sum(m, axis=1, keepdims=True)

    @pl.when(i == NB - 1)
    def _():
        cv = cnt[...]
        c2 = jnp.maximum(jnp.concatenate([cv, cv], axis=1), 1.0)
        pooled = acc[...] / c2
        out[...] = (jnp.dot(pooled, wfct[...], preferred_element_type=jnp.float32)
                    + bfc[...])


_tc_dense = pl.pallas_call(
    _tc_body,
    grid=(NB,),
    in_specs=[
        pl.BlockSpec((1, BN, HALF), lambda i: (0, i, 0)),
        pl.BlockSpec((1, BN, HALF), lambda i: (1, i, 0)),
        pl.BlockSpec((BN, F), lambda i: (i, 0)),
        pl.BlockSpec((1, 1, BN), lambda i: (i, 0, 0)),
        pl.BlockSpec((HALF, F), lambda i: (0, 0)),
        pl.BlockSpec((HALF, F), lambda i: (0, 0)),
        pl.BlockSpec((F, F), lambda i: (0, 0)),
        pl.BlockSpec((1, F), lambda i: (0, 0)),
        pl.BlockSpec((F, C), lambda i: (0, 0)),
        pl.BlockSpec((1, C), lambda i: (0, 0)),
    ],
    out_specs=pl.BlockSpec((G, C), lambda i: (0, 0)),
    out_shape=jax.ShapeDtypeStruct((G, C), jnp.float32),
    scratch_shapes=[
        pltpu.VMEM((G, F), jnp.float32),
        pltpu.VMEM((G, HALF), jnp.float32),
    ],
)


def kernel(x, edge_index, batch, W_rel, b_rel, W_root, W_fc, b_fc):
    src = edge_index[0].astype(jnp.int32)
    dst = edge_index[1].astype(jnp.int32)

    pad = E_TILE_PAD - E_TILE
    src_t = jnp.concatenate(
        [src.reshape(NS, E_TILE), jnp.zeros((NS, pad), jnp.int32)], axis=1)
    dst_t = jnp.concatenate(
        [dst.reshape(NS, E_TILE), jnp.full((NS, pad), N, jnp.int32)], axis=1)
    src3 = jnp.stack([2 * src_t, 2 * src_t + 1]).reshape(NC, NS, NCHUNK, CHUNK)
    dst3 = dst_t.reshape(NS, NCHUNK, CHUNK)
    xh = x.reshape(2 * N, HALF)

    agg = _sc_scatter()(xh, src3, dst3)

    batch3 = batch.astype(jnp.int32).reshape(NB, 1, BN)
    return _tc_dense(
        agg, agg, x, batch3,
        W_rel.T[:HALF].astype(jnp.float32),
        W_rel.T[HALF:].astype(jnp.float32),
        W_root.T.astype(jnp.float32),
        b_rel.reshape(1, F),
        W_fc.T.astype(jnp.float32),
        b_fc.reshape(1, C),
    )
